# baseline (speedup 1.0000x reference)
import functools

import jax
import jax.numpy as jnp
from jax import lax
from jax.experimental import pallas as pl
from jax.experimental.pallas import tpu as pltpu
from jax.experimental.pallas import tpu_sc as plsc

CH = 128
NW = 32
BE = 3200


GK = 6
GKS = 4
CHS = 64
PAD_CH = 24


def _worker_range(w, nch):
    lo = ((w * nch) // NW) // 8 * 8
    hi = jnp.where(w == NW - 1, nch, (((w + 1) * nch) // NW) // 8 * 8)
    return lo, hi


SPMEM_BUDGET = 1_900_000


def _sc_gather(table, idx2d):
    nch, ch = idx2d.shape
    nch -= PAD_CH
    t, d = table.shape
    n_rows = nch * ch
    bufrows = (nch // NW + 8 + 7) // 8 * 8
    st0 = (t // 16) // 8 * 8
    stl = t - 15 * st0

    @functools.partial(
        pl.kernel,
        out_type=jax.ShapeDtypeStruct((n_rows, d), jnp.float32),
        mesh=plsc.VectorSubcoreMesh(core_axis_name="c", subcore_axis_name="s"),
        scratch_types=[
            pltpu.VMEM((bufrows, ch), jnp.int32),
            pltpu.VMEM((2 * ch, d), jnp.float32),
            pltpu.VMEM_SHARED((t, d), jnp.float32),
            pltpu.SemaphoreType.DMA,
            pltpu.SemaphoreType.DMA,
        ],
    )
    def k(table_hbm, idx_hbm, out_hbm, idx_v, rows_v, tab_sh, semg, semw):
        s = lax.axis_index("s")
        w = s * 2 + lax.axis_index("c")
        lo, hi = _worker_range(w, nch)
        n = hi - lo

        @pl.when(s < 15)
        def _():
            pltpu.sync_copy(table_hbm.at[pl.ds(s * st0, st0)],
                            tab_sh.at[pl.ds(s * st0, st0)])

        @pl.when(s == 15)
        def _():
            pltpu.sync_copy(table_hbm.at[pl.ds(15 * st0, stl)],
                            tab_sh.at[pl.ds(15 * st0, stl)])

        pltpu.sync_copy(idx_hbm.at[pl.ds(lo, bufrows)], idx_v)
        plsc.subcore_barrier()

        def drain_one_write():
            pltpu.make_async_copy(
                out_hbm.at[pl.ds(0, ch)], rows_v.at[pl.ds(0, ch)],
                semw).wait()

        def body(i, carry):
            for b in range(2):
                j = lo + 2 * i + b

                @pl.when(i > 0)
                def _():
                    drain_one_write()

                @pl.when(j < hi)
                def _():
                    pltpu.async_copy(
                        tab_sh.at[idx_v.at[2 * i + b]],
                        rows_v.at[pl.ds(b * ch, ch)], semg).wait()
                    pltpu.async_copy(
                        rows_v.at[pl.ds(b * ch, ch)],
                        out_hbm.at[pl.ds(j * ch, ch)], semw)

            return carry

        lax.fori_loop(0, (n + 1) // 2, body, 0)

        @pl.when(n >= 1)
        def _():
            drain_one_write()

        @pl.when(jnp.logical_and(n >= 2, n % 2 == 0))
        def _():
            drain_one_write()

    return k(table, idx2d)


def _sc_scatter(msg, idx2d, zrows):
    nch, ch = idx2d.shape
    nch -= PAD_CH
    d = msg.shape[1]
    np_rows = zrows.shape[0] * 16
    bufrows = (nch // NW + 8 + 7) // 8 * 8

    @functools.partial(
        pl.kernel,
        out_type=jax.ShapeDtypeStruct((2 * np_rows, d), jnp.float32),
        mesh=plsc.VectorSubcoreMesh(core_axis_name="c", subcore_axis_name="s"),
        scratch_types=[
            pltpu.VMEM((bufrows, ch), jnp.int32),
            pltpu.VMEM((GKS * ch, d), jnp.float32),
            pltpu.VMEM_SHARED((np_rows, d), jnp.float32),
            pltpu.SemaphoreType.DMA,
        ],
    )
    def k(msg_hbm, idx_hbm, z_hbm, out_hbm, idx_v, rows_v, acc_sh, sem):
        c = lax.axis_index("c")
        s = lax.axis_index("s")
        stripe = np_rows // 16
        pltpu.sync_copy(z_hbm, acc_sh.at[pl.ds(s * stripe, stripe)])
        plsc.subcore_barrier()
        w = s * 2 + c
        lo, hi = _worker_range(w, nch)
        pltpu.sync_copy(idx_hbm.at[pl.ds(lo, bufrows)], idx_v)

        def body(i, carry):
            j = lo + i * GKS

            @pl.when(i > 0)
            def _():
                pltpu.make_async_copy(
                    msg_hbm.at[pl.ds(0, GKS * ch)], rows_v, sem).wait()

            @pl.when(j + GKS <= hi)
            def _():
                pltpu.sync_copy(msg_hbm.at[pl.ds(j * ch, GKS * ch)], rows_v)
                for b in range(GKS):
                    pltpu.async_copy(
                        rows_v.at[pl.ds(b * ch, ch)],
                        acc_sh.at[idx_v.at[i * GKS + b]], sem, add=True)

            @pl.when(j + GKS > hi)
            def _():
                for b in range(GKS):
                    @pl.when(j + b < hi)
                    def _(b=b):
                        pltpu.sync_copy(
                            msg_hbm.at[pl.ds((j + b) * ch, ch)],
                            rows_v.at[pl.ds(b * ch, ch)])
                        pltpu.sync_copy(
                            rows_v.at[pl.ds(b * ch, ch)],
                            acc_sh.at[idx_v.at[i * GKS + b]], add=True)

            return carry

        lax.fori_loop(0, (hi - lo + GKS - 1) // GKS, body, 0)

        @pl.when(jnp.logical_and(hi > lo, (hi - lo) % GKS == 0))
        def _():
            pltpu.make_async_copy(
                msg_hbm.at[pl.ds(0, GKS * ch)], rows_v, sem).wait()

        plsc.subcore_barrier()
        pltpu.sync_copy(
            acc_sh.at[pl.ds(s * stripe, stripe)],
            out_hbm.at[pl.ds(c * np_rows + s * stripe, stripe)],
        )

    return k(msg, idx2d, zrows)


def _prep_body(te_ref, me_ref, w1t_ref, w1m_ref, b1_ref, rt_ref, rm_ref,
               out_ref):
    tpre = jax.lax.dot_general(
        te_ref[...], w1t_ref[...], (((1,), (0,)), ((), ())),
        preferred_element_type=jnp.float32)
    mpre = jax.lax.dot_general(
        me_ref[...], w1m_ref[...], (((1,), (0,)), ((), ())),
        preferred_element_type=jnp.float32)
    out_ref[...] = (
        jax.lax.dot_general(rt_ref[...], tpre, (((1,), (0,)), ((), ())),
                            preferred_element_type=jnp.float32)
        + jax.lax.dot_general(rm_ref[...], mpre, (((1,), (0,)), ((), ())),
                              preferred_element_type=jnp.float32)
        + b1_ref[...])


def _prep(tissue_emb, metagene_emb, w1t, w1m, b1):
    nt, d = tissue_emb.shape
    nm = metagene_emb.shape[0]
    rows = jnp.arange(nt * nm, dtype=jnp.int32)
    rt = jax.nn.one_hot(rows // nm, nt, dtype=jnp.float32)
    rm = jax.nn.one_hot(rows % nm, nm, dtype=jnp.float32)
    return pl.pallas_call(
        _prep_body,
        out_shape=jax.ShapeDtypeStruct((nt * nm, d), jnp.float32),
    )(tissue_emb, metagene_emb, w1t, w1m, b1, rt, rm)


def _xp_body(x_ref, w_ref, out_ref):
    out_ref[...] = jax.lax.dot_general(
        x_ref[...], w_ref[...], (((1,), (0,)), ((), ())),
        preferred_element_type=jnp.float32)


def _xp(x, w1p):
    n, d = x.shape
    bn = 1000 if n % 1000 == 0 else n
    return pl.pallas_call(
        _xp_body,
        grid=(n // bn,),
        in_specs=[
            pl.BlockSpec((bn, d), lambda i: (i, 0)),
            pl.BlockSpec((d, d), lambda i: (0, 0)),
        ],
        out_specs=pl.BlockSpec((bn, d), lambda i: (i, 0)),
        out_shape=jax.ShapeDtypeStruct((n, d), jnp.float32),
        compiler_params=pltpu.CompilerParams(dimension_semantics=("parallel",)),
    )(x, w1p)


def _edge_body(xg_ref, gtm_ref, ea_ref, w1a_ref, w2_ref, b2_ref, we_ref,
               be_ref, msg_ref, eao_ref):
    ea = ea_ref[...]
    h = xg_ref[...] + gtm_ref[...] + jax.lax.dot_general(
        ea, w1a_ref[...], (((0,), (0,)), ((), ())),
        preferred_element_type=jnp.float32)
    h = jnp.maximum(h, 0.0)
    msg = jax.lax.dot_general(
        h, w2_ref[...], (((1,), (0,)), ((), ())),
        preferred_element_type=jnp.float32) + b2_ref[...]
    msg = jnp.maximum(msg, 0.0)
    msg_ref[...] = msg
    eao_ref[...] = ea + jax.lax.dot_general(
        we_ref[...], msg, (((0,), (1,)), ((), ())),
        preferred_element_type=jnp.float32) + be_ref[...]


def _edge(xg, gtm, ea_t, w1a, w2, b2r, we, ber, goff=0):
    e, d = xg.shape
    da = ea_t.shape[0]
    be_blk = BE if e % BE == 0 else e
    gob = goff // be_blk
    return pl.pallas_call(
        _edge_body,
        grid=(e // be_blk,),
        in_specs=[
            pl.BlockSpec((be_blk, d), lambda i: (i, 0)),
            pl.BlockSpec((be_blk, d), lambda i: (i + gob, 0)),
            pl.BlockSpec((da, be_blk), lambda i: (0, i)),
            pl.BlockSpec((da, d), lambda i: (0, 0)),
            pl.BlockSpec((d, d), lambda i: (0, 0)),
            pl.BlockSpec((1, d), lambda i: (0, 0)),
            pl.BlockSpec((d, da), lambda i: (0, 0)),
            pl.BlockSpec((da, 1), lambda i: (0, 0)),
        ],
        out_specs=[
            pl.BlockSpec((be_blk, d), lambda i: (i, 0)),
            pl.BlockSpec((da, be_blk), lambda i: (0, i)),
        ],
        out_shape=[
            jax.ShapeDtypeStruct((e, d), jnp.float32),
            jax.ShapeDtypeStruct((da, e), jnp.float32),
        ],
        compiler_params=pltpu.CompilerParams(dimension_semantics=("parallel",)),
    )(xg, gtm, ea_t, w1a, w2, b2r, we, ber)


def _upd_body(agg_ref, aggb_ref, x_ref, wu_ref, bu_ref, w1p_ref, xn_ref,
              xpn_ref):
    agg = (agg_ref[0] + agg_ref[1]) + (aggb_ref[0] + aggb_ref[1])
    upd = jax.lax.dot_general(
        agg, wu_ref[...], (((1,), (0,)), ((), ())),
        preferred_element_type=jnp.float32) + bu_ref[...]
    xn = x_ref[...] + jnp.maximum(upd, 0.0)
    xn_ref[...] = xn
    xpn_ref[...] = jax.lax.dot_general(
        xn, w1p_ref[...], (((1,), (0,)), ((), ())),
        preferred_element_type=jnp.float32)


def _upd(agg2, agg2b, x, wu, bur, w1p):
    n, d = x.shape
    bn = 1000 if n % 1000 == 0 else n
    return pl.pallas_call(
        _upd_body,
        grid=(n // bn,),
        in_specs=[
            pl.BlockSpec((2, bn, d), lambda i: (0, i, 0)),
            pl.BlockSpec((2, bn, d), lambda i: (0, i, 0)),
            pl.BlockSpec((bn, d), lambda i: (i, 0)),
            pl.BlockSpec((d, d), lambda i: (0, 0)),
            pl.BlockSpec((1, d), lambda i: (0, 0)),
            pl.BlockSpec((d, d), lambda i: (0, 0)),
        ],
        out_specs=[
            pl.BlockSpec((bn, d), lambda i: (i, 0)),
            pl.BlockSpec((bn, d), lambda i: (i, 0)),
        ],
        out_shape=[
            jax.ShapeDtypeStruct((n, d), jnp.float32),
            jax.ShapeDtypeStruct((n, d), jnp.float32),
        ],
        compiler_params=pltpu.CompilerParams(dimension_semantics=("parallel",)),
    )(agg2, agg2b, x, wu, bur, w1p)


def kernel(patient_idx, tissue_idx, metagene_idx, hyperedge_attr,
           patient_features, tissue_emb, metagene_emb,
           W1, b1, W2, b2, Wu, bu, We, be):
    e, da = hyperedge_attr.shape
    npat, d = patient_features.shape
    nm = metagene_emb.shape[0]
    w1p, w1t, w1m, w1a = W1[:d], W1[d:2 * d], W1[2 * d:3 * d], W1[3 * d:]
    pidx = patient_idx.astype(jnp.int32)
    cidx = (tissue_idx.astype(jnp.int32) * nm + metagene_idx.astype(jnp.int32))
    half = e // 2

    def _chunked(ix):
        return jnp.pad(ix.reshape(-1, CH), ((0, PAD_CH), (0, 0)))

    pidx2 = [_chunked(pidx[:half]), _chunked(pidx[half:])]
    cidx2 = _chunked(cidx)

    def _chunked_s(ix):
        return jnp.pad(ix.reshape(-1, CHS), ((0, PAD_CH), (0, 0)))

    pidx2s = [_chunked_s(pidx[:half]), _chunked_s(pidx[half:])]
    ea_t = [hyperedge_attr[:half].T, hyperedge_attr[half:].T]
    b1r = b1.reshape(1, d)
    b2r = b2.reshape(1, d)
    ber = be.reshape(da, 1)
    bur = bu.reshape(1, d)
    stripe = ((npat + 15) // 16 + 7) // 8 * 8
    zrows = jnp.zeros((stripe, d), jnp.float32)
    npad = stripe * 16

    ctab = _prep(tissue_emb, metagene_emb, w1t, w1m, b1r)
    gtm = _sc_gather(ctab, cidx2)
    x = patient_features
    xp = _xp(x, w1p)
    for _ in range(3):
        xg = [_sc_gather(xp, pidx2[0]), _sc_gather(xp, pidx2[1])]
        agg2 = [None, None]
        for hf in range(2):
            msg, ea_t[hf] = _edge(xg[hf], gtm, ea_t[hf], w1a, W2, b2r,
                                  We, ber, goff=hf * half)
            agg2[hf] = _sc_scatter(msg, pidx2s[hf], zrows)
        agg2 = [a.reshape(2, npad, d)[:, :npat] for a in agg2]
        x, xp = _upd(agg2[0], agg2[1], x, Wu, bur, w1p)
    return x


# speedup vs baseline: 1.0036x; 1.0036x over previous
import functools

import jax
import jax.numpy as jnp
from jax import lax
from jax.experimental import pallas as pl
from jax.experimental.pallas import tpu as pltpu
from jax.experimental.pallas import tpu_sc as plsc

CH = 128
NW = 32
BE = 3200


GK = 6
GKS = 2
CHS = 128
PAD_CH = 24


def _worker_range(w, nch):
    lo = ((w * nch) // NW) // 8 * 8
    hi = jnp.where(w == NW - 1, nch, (((w + 1) * nch) // NW) // 8 * 8)
    return lo, hi


SPMEM_BUDGET = 1_900_000


def _sc_gather(table, idx2d):
    nch, ch = idx2d.shape
    nch -= PAD_CH
    t, d = table.shape
    n_rows = nch * ch
    bufrows = (nch // NW + 8 + 7) // 8 * 8
    st0 = (t // 16) // 8 * 8
    stl = t - 15 * st0

    @functools.partial(
        pl.kernel,
        out_type=jax.ShapeDtypeStruct((n_rows, d), jnp.float32),
        mesh=plsc.VectorSubcoreMesh(core_axis_name="c", subcore_axis_name="s"),
        scratch_types=[
            pltpu.VMEM((bufrows, ch), jnp.int32),
            pltpu.VMEM((2 * ch, d), jnp.float32),
            pltpu.VMEM_SHARED((t, d), jnp.float32),
            pltpu.SemaphoreType.DMA,
            pltpu.SemaphoreType.DMA,
        ],
    )
    def k(table_hbm, idx_hbm, out_hbm, idx_v, rows_v, tab_sh, semg, semw):
        s = lax.axis_index("s")
        w = s * 2 + lax.axis_index("c")
        lo, hi = _worker_range(w, nch)
        n = hi - lo

        @pl.when(s < 15)
        def _():
            pltpu.sync_copy(table_hbm.at[pl.ds(s * st0, st0)],
                            tab_sh.at[pl.ds(s * st0, st0)])

        @pl.when(s == 15)
        def _():
            pltpu.sync_copy(table_hbm.at[pl.ds(15 * st0, stl)],
                            tab_sh.at[pl.ds(15 * st0, stl)])

        pltpu.sync_copy(idx_hbm.at[pl.ds(lo, bufrows)], idx_v)
        plsc.subcore_barrier()

        def drain_one_write():
            pltpu.make_async_copy(
                out_hbm.at[pl.ds(0, ch)], rows_v.at[pl.ds(0, ch)],
                semw).wait()

        def body(i, carry):
            for b in range(2):
                j = lo + 2 * i + b

                @pl.when(i > 0)
                def _():
                    drain_one_write()

                @pl.when(j < hi)
                def _():
                    pltpu.async_copy(
                        tab_sh.at[idx_v.at[2 * i + b]],
                        rows_v.at[pl.ds(b * ch, ch)], semg).wait()
                    pltpu.async_copy(
                        rows_v.at[pl.ds(b * ch, ch)],
                        out_hbm.at[pl.ds(j * ch, ch)], semw)

            return carry

        lax.fori_loop(0, (n + 1) // 2, body, 0)

        @pl.when(n >= 1)
        def _():
            drain_one_write()

        @pl.when(jnp.logical_and(n >= 2, n % 2 == 0))
        def _():
            drain_one_write()

    return k(table, idx2d)


def _sc_scatter(msg, idx2d, zrows):
    nch, ch = idx2d.shape
    nch -= PAD_CH
    d = msg.shape[1]
    np_rows = zrows.shape[0] * 16
    bufrows = (nch // NW + 8 + 7) // 8 * 8

    @functools.partial(
        pl.kernel,
        out_type=jax.ShapeDtypeStruct((2 * np_rows, d), jnp.float32),
        mesh=plsc.VectorSubcoreMesh(core_axis_name="c", subcore_axis_name="s"),
        scratch_types=[
            pltpu.VMEM((bufrows, ch), jnp.int32),
            pltpu.VMEM((GKS * ch, d), jnp.float32),
            pltpu.VMEM_SHARED((np_rows, d), jnp.float32),
            pltpu.SemaphoreType.DMA,
        ],
    )
    def k(msg_hbm, idx_hbm, z_hbm, out_hbm, idx_v, rows_v, acc_sh, sem):
        c = lax.axis_index("c")
        s = lax.axis_index("s")
        stripe = np_rows // 16
        pltpu.sync_copy(z_hbm, acc_sh.at[pl.ds(s * stripe, stripe)])
        plsc.subcore_barrier()
        w = s * 2 + c
        lo, hi = _worker_range(w, nch)
        pltpu.sync_copy(idx_hbm.at[pl.ds(lo, bufrows)], idx_v)

        def body(i, carry):
            j = lo + i * GKS

            @pl.when(i > 0)
            def _():
                pltpu.make_async_copy(
                    msg_hbm.at[pl.ds(0, GKS * ch)], rows_v, sem).wait()

            @pl.when(j + GKS <= hi)
            def _():
                pltpu.sync_copy(msg_hbm.at[pl.ds(j * ch, GKS * ch)], rows_v)
                for b in range(GKS):
                    pltpu.async_copy(
                        rows_v.at[pl.ds(b * ch, ch)],
                        acc_sh.at[idx_v.at[i * GKS + b]], sem, add=True)

            @pl.when(j + GKS > hi)
            def _():
                for b in range(GKS):
                    @pl.when(j + b < hi)
                    def _(b=b):
                        pltpu.sync_copy(
                            msg_hbm.at[pl.ds((j + b) * ch, ch)],
                            rows_v.at[pl.ds(b * ch, ch)])
                        pltpu.sync_copy(
                            rows_v.at[pl.ds(b * ch, ch)],
                            acc_sh.at[idx_v.at[i * GKS + b]], add=True)

            return carry

        lax.fori_loop(0, (hi - lo + GKS - 1) // GKS, body, 0)

        @pl.when(jnp.logical_and(hi > lo, (hi - lo) % GKS == 0))
        def _():
            pltpu.make_async_copy(
                msg_hbm.at[pl.ds(0, GKS * ch)], rows_v, sem).wait()

        plsc.subcore_barrier()
        pltpu.sync_copy(
            acc_sh.at[pl.ds(s * stripe, stripe)],
            out_hbm.at[pl.ds(c * np_rows + s * stripe, stripe)],
        )

    return k(msg, idx2d, zrows)


def _prep_body(te_ref, me_ref, w1t_ref, w1m_ref, b1_ref, rt_ref, rm_ref,
               out_ref):
    tpre = jax.lax.dot_general(
        te_ref[...], w1t_ref[...], (((1,), (0,)), ((), ())),
        preferred_element_type=jnp.float32)
    mpre = jax.lax.dot_general(
        me_ref[...], w1m_ref[...], (((1,), (0,)), ((), ())),
        preferred_element_type=jnp.float32)
    out_ref[...] = (
        jax.lax.dot_general(rt_ref[...], tpre, (((1,), (0,)), ((), ())),
                            preferred_element_type=jnp.float32)
        + jax.lax.dot_general(rm_ref[...], mpre, (((1,), (0,)), ((), ())),
                              preferred_element_type=jnp.float32)
        + b1_ref[...])


def _prep(tissue_emb, metagene_emb, w1t, w1m, b1):
    nt, d = tissue_emb.shape
    nm = metagene_emb.shape[0]
    rows = jnp.arange(nt * nm, dtype=jnp.int32)
    rt = jax.nn.one_hot(rows // nm, nt, dtype=jnp.float32)
    rm = jax.nn.one_hot(rows % nm, nm, dtype=jnp.float32)
    return pl.pallas_call(
        _prep_body,
        out_shape=jax.ShapeDtypeStruct((nt * nm, d), jnp.float32),
    )(tissue_emb, metagene_emb, w1t, w1m, b1, rt, rm)


def _xp_body(x_ref, w_ref, out_ref):
    out_ref[...] = jax.lax.dot_general(
        x_ref[...], w_ref[...], (((1,), (0,)), ((), ())),
        preferred_element_type=jnp.float32)


def _xp(x, w1p):
    n, d = x.shape
    bn = 1000 if n % 1000 == 0 else n
    return pl.pallas_call(
        _xp_body,
        grid=(n // bn,),
        in_specs=[
            pl.BlockSpec((bn, d), lambda i: (i, 0)),
            pl.BlockSpec((d, d), lambda i: (0, 0)),
        ],
        out_specs=pl.BlockSpec((bn, d), lambda i: (i, 0)),
        out_shape=jax.ShapeDtypeStruct((n, d), jnp.float32),
        compiler_params=pltpu.CompilerParams(dimension_semantics=("parallel",)),
    )(x, w1p)


def _edge_body(xg_ref, gtm_ref, ea_ref, w1a_ref, w2_ref, b2_ref, we_ref,
               be_ref, msg_ref, eao_ref):
    ea = ea_ref[...]
    h = xg_ref[...] + gtm_ref[...] + jax.lax.dot_general(
        ea, w1a_ref[...], (((0,), (0,)), ((), ())),
        preferred_element_type=jnp.float32)
    h = jnp.maximum(h, 0.0)
    msg = jax.lax.dot_general(
        h.astype(jnp.bfloat16), w2_ref[...].astype(jnp.bfloat16),
        (((1,), (0,)), ((), ())),
        preferred_element_type=jnp.float32) + b2_ref[...]
    msg = jnp.maximum(msg, 0.0)
    msg_ref[...] = msg
    eao_ref[...] = ea + jax.lax.dot_general(
        we_ref[...], msg, (((0,), (1,)), ((), ())),
        preferred_element_type=jnp.float32) + be_ref[...]


def _edge(xg, gtm, ea_t, w1a, w2, b2r, we, ber, goff=0):
    e, d = xg.shape
    da = ea_t.shape[0]
    be_blk = BE if e % BE == 0 else e
    gob = goff // be_blk
    return pl.pallas_call(
        _edge_body,
        grid=(e // be_blk,),
        in_specs=[
            pl.BlockSpec((be_blk, d), lambda i: (i, 0)),
            pl.BlockSpec((be_blk, d), lambda i: (i + gob, 0)),
            pl.BlockSpec((da, be_blk), lambda i: (0, i)),
            pl.BlockSpec((da, d), lambda i: (0, 0)),
            pl.BlockSpec((d, d), lambda i: (0, 0)),
            pl.BlockSpec((1, d), lambda i: (0, 0)),
            pl.BlockSpec((d, da), lambda i: (0, 0)),
            pl.BlockSpec((da, 1), lambda i: (0, 0)),
        ],
        out_specs=[
            pl.BlockSpec((be_blk, d), lambda i: (i, 0)),
            pl.BlockSpec((da, be_blk), lambda i: (0, i)),
        ],
        out_shape=[
            jax.ShapeDtypeStruct((e, d), jnp.float32),
            jax.ShapeDtypeStruct((da, e), jnp.float32),
        ],
        compiler_params=pltpu.CompilerParams(dimension_semantics=("parallel",)),
    )(xg, gtm, ea_t, w1a, w2, b2r, we, ber)


def _upd_body(agg_ref, aggb_ref, x_ref, wu_ref, bu_ref, w1p_ref, xn_ref,
              xpn_ref):
    agg = (agg_ref[0] + agg_ref[1]) + (aggb_ref[0] + aggb_ref[1])
    upd = jax.lax.dot_general(
        agg, wu_ref[...], (((1,), (0,)), ((), ())),
        preferred_element_type=jnp.float32) + bu_ref[...]
    xn = x_ref[...] + jnp.maximum(upd, 0.0)
    xn_ref[...] = xn
    xpn_ref[...] = jax.lax.dot_general(
        xn, w1p_ref[...], (((1,), (0,)), ((), ())),
        preferred_element_type=jnp.float32)


def _upd(agg2, agg2b, x, wu, bur, w1p):
    n, d = x.shape
    bn = 1000 if n % 1000 == 0 else n
    return pl.pallas_call(
        _upd_body,
        grid=(n // bn,),
        in_specs=[
            pl.BlockSpec((2, bn, d), lambda i: (0, i, 0)),
            pl.BlockSpec((2, bn, d), lambda i: (0, i, 0)),
            pl.BlockSpec((bn, d), lambda i: (i, 0)),
            pl.BlockSpec((d, d), lambda i: (0, 0)),
            pl.BlockSpec((1, d), lambda i: (0, 0)),
            pl.BlockSpec((d, d), lambda i: (0, 0)),
        ],
        out_specs=[
            pl.BlockSpec((bn, d), lambda i: (i, 0)),
            pl.BlockSpec((bn, d), lambda i: (i, 0)),
        ],
        out_shape=[
            jax.ShapeDtypeStruct((n, d), jnp.float32),
            jax.ShapeDtypeStruct((n, d), jnp.float32),
        ],
        compiler_params=pltpu.CompilerParams(dimension_semantics=("parallel",)),
    )(agg2, agg2b, x, wu, bur, w1p)


def kernel(patient_idx, tissue_idx, metagene_idx, hyperedge_attr,
           patient_features, tissue_emb, metagene_emb,
           W1, b1, W2, b2, Wu, bu, We, be):
    e, da = hyperedge_attr.shape
    npat, d = patient_features.shape
    nm = metagene_emb.shape[0]
    w1p, w1t, w1m, w1a = W1[:d], W1[d:2 * d], W1[2 * d:3 * d], W1[3 * d:]
    pidx = patient_idx.astype(jnp.int32)
    cidx = (tissue_idx.astype(jnp.int32) * nm + metagene_idx.astype(jnp.int32))
    half = e // 2

    def _chunked(ix):
        return jnp.pad(ix.reshape(-1, CH), ((0, PAD_CH), (0, 0)))

    pidx2 = [_chunked(pidx[:half]), _chunked(pidx[half:])]
    cidx2 = _chunked(cidx)

    def _chunked_s(ix):
        return jnp.pad(ix.reshape(-1, CHS), ((0, PAD_CH), (0, 0)))

    pidx2s = [_chunked_s(pidx[:half]), _chunked_s(pidx[half:])]
    ea_t = [hyperedge_attr[:half].T, hyperedge_attr[half:].T]
    b1r = b1.reshape(1, d)
    b2r = b2.reshape(1, d)
    ber = be.reshape(da, 1)
    bur = bu.reshape(1, d)
    stripe = ((npat + 15) // 16 + 7) // 8 * 8
    zrows = jnp.zeros((stripe, d), jnp.float32)
    npad = stripe * 16

    ctab = _prep(tissue_emb, metagene_emb, w1t, w1m, b1r)
    gtm = _sc_gather(ctab, cidx2)
    x = patient_features
    xp = _xp(x, w1p)
    for _ in range(3):
        xg = [_sc_gather(xp, pidx2[0]), _sc_gather(xp, pidx2[1])]
        agg2 = [None, None]
        for hf in range(2):
            msg, ea_t[hf] = _edge(xg[hf], gtm, ea_t[hf], w1a, W2, b2r,
                                  We, ber, goff=hf * half)
            agg2[hf] = _sc_scatter(msg, pidx2s[hf], zrows)
        agg2 = [a.reshape(2, npad, d)[:, :npat] for a in agg2]
        x, xp = _upd(agg2[0], agg2[1], x, Wu, bur, w1p)
    return x


# speedup vs baseline: 1.0043x; 1.0007x over previous
import functools

import jax
import jax.numpy as jnp
from jax import lax
from jax.experimental import pallas as pl
from jax.experimental.pallas import tpu as pltpu
from jax.experimental.pallas import tpu_sc as plsc

CH = 128
NW = 32
BE = 3200


GK = 6
GKS = 2
CHS = 128
PAD_CH = 24


def _worker_range(w, nch):
    lo = ((w * nch) // NW) // 8 * 8
    hi = jnp.where(w == NW - 1, nch, (((w + 1) * nch) // NW) // 8 * 8)
    return lo, hi


SPMEM_BUDGET = 1_900_000


def _sc_gather(table, idx2d):
    nch, ch = idx2d.shape
    nch -= PAD_CH
    t, d = table.shape
    n_rows = nch * ch
    bufrows = (nch // NW + 8 + 7) // 8 * 8
    st0 = (t // 16) // 8 * 8
    stl = t - 15 * st0

    @functools.partial(
        pl.kernel,
        out_type=jax.ShapeDtypeStruct((n_rows, d), jnp.float32),
        mesh=plsc.VectorSubcoreMesh(core_axis_name="c", subcore_axis_name="s"),
        scratch_types=[
            pltpu.VMEM((bufrows, ch), jnp.int32),
            pltpu.VMEM((2 * ch, d), jnp.float32),
            pltpu.VMEM_SHARED((t, d), jnp.float32),
            pltpu.SemaphoreType.DMA,
            pltpu.SemaphoreType.DMA,
        ],
    )
    def k(table_hbm, idx_hbm, out_hbm, idx_v, rows_v, tab_sh, semg, semw):
        s = lax.axis_index("s")
        w = s * 2 + lax.axis_index("c")
        lo, hi = _worker_range(w, nch)
        n = hi - lo

        @pl.when(s < 15)
        def _():
            pltpu.sync_copy(table_hbm.at[pl.ds(s * st0, st0)],
                            tab_sh.at[pl.ds(s * st0, st0)])

        @pl.when(s == 15)
        def _():
            pltpu.sync_copy(table_hbm.at[pl.ds(15 * st0, stl)],
                            tab_sh.at[pl.ds(15 * st0, stl)])

        pltpu.sync_copy(idx_hbm.at[pl.ds(lo, bufrows)], idx_v)
        plsc.subcore_barrier()

        def drain_one_write():
            pltpu.make_async_copy(
                out_hbm.at[pl.ds(0, ch)], rows_v.at[pl.ds(0, ch)],
                semw).wait()

        def body(i, carry):
            for b in range(2):
                j = lo + 2 * i + b

                @pl.when(i > 0)
                def _():
                    drain_one_write()

                @pl.when(j < hi)
                def _():
                    pltpu.async_copy(
                        tab_sh.at[idx_v.at[2 * i + b]],
                        rows_v.at[pl.ds(b * ch, ch)], semg).wait()
                    pltpu.async_copy(
                        rows_v.at[pl.ds(b * ch, ch)],
                        out_hbm.at[pl.ds(j * ch, ch)], semw)

            return carry

        lax.fori_loop(0, (n + 1) // 2, body, 0)

        @pl.when(n >= 1)
        def _():
            drain_one_write()

        @pl.when(jnp.logical_and(n >= 2, n % 2 == 0))
        def _():
            drain_one_write()

    return k(table, idx2d)


def _sc_scatter(msg, idx2d, zrows):
    nch, ch = idx2d.shape
    nch -= PAD_CH
    d = msg.shape[1]
    np_rows = zrows.shape[0] * 16
    bufrows = (nch // NW + 8 + 7) // 8 * 8

    @functools.partial(
        pl.kernel,
        out_type=jax.ShapeDtypeStruct((2 * np_rows, d), jnp.float32),
        mesh=plsc.VectorSubcoreMesh(core_axis_name="c", subcore_axis_name="s"),
        scratch_types=[
            pltpu.VMEM((bufrows, ch), jnp.int32),
            pltpu.VMEM((GKS * ch, d), jnp.float32),
            pltpu.VMEM_SHARED((np_rows, d), jnp.float32),
            pltpu.SemaphoreType.DMA,
        ],
    )
    def k(msg_hbm, idx_hbm, z_hbm, out_hbm, idx_v, rows_v, acc_sh, sem):
        c = lax.axis_index("c")
        s = lax.axis_index("s")
        stripe = np_rows // 16
        pltpu.sync_copy(z_hbm, acc_sh.at[pl.ds(s * stripe, stripe)])
        plsc.subcore_barrier()
        w = s * 2 + c
        lo, hi = _worker_range(w, nch)
        pltpu.sync_copy(idx_hbm.at[pl.ds(lo, bufrows)], idx_v)

        def body(i, carry):
            j = lo + i * GKS

            @pl.when(i > 0)
            def _():
                pltpu.make_async_copy(
                    msg_hbm.at[pl.ds(0, GKS * ch)], rows_v, sem).wait()

            @pl.when(j + GKS <= hi)
            def _():
                pltpu.sync_copy(msg_hbm.at[pl.ds(j * ch, GKS * ch)], rows_v)
                for b in range(GKS):
                    pltpu.async_copy(
                        rows_v.at[pl.ds(b * ch, ch)],
                        acc_sh.at[idx_v.at[i * GKS + b]], sem, add=True)

            @pl.when(j + GKS > hi)
            def _():
                for b in range(GKS):
                    @pl.when(j + b < hi)
                    def _(b=b):
                        pltpu.sync_copy(
                            msg_hbm.at[pl.ds((j + b) * ch, ch)],
                            rows_v.at[pl.ds(b * ch, ch)])
                        pltpu.sync_copy(
                            rows_v.at[pl.ds(b * ch, ch)],
                            acc_sh.at[idx_v.at[i * GKS + b]], add=True)

            return carry

        lax.fori_loop(0, (hi - lo + GKS - 1) // GKS, body, 0)

        @pl.when(jnp.logical_and(hi > lo, (hi - lo) % GKS == 0))
        def _():
            pltpu.make_async_copy(
                msg_hbm.at[pl.ds(0, GKS * ch)], rows_v, sem).wait()

        plsc.subcore_barrier()
        pltpu.sync_copy(
            acc_sh.at[pl.ds(s * stripe, stripe)],
            out_hbm.at[pl.ds(c * np_rows + s * stripe, stripe)],
        )

    return k(msg, idx2d, zrows)


def _prep_body(te_ref, me_ref, w1t_ref, w1m_ref, b1_ref, rt_ref, rm_ref,
               out_ref):
    tpre = jax.lax.dot_general(
        te_ref[...], w1t_ref[...], (((1,), (0,)), ((), ())),
        preferred_element_type=jnp.float32)
    mpre = jax.lax.dot_general(
        me_ref[...], w1m_ref[...], (((1,), (0,)), ((), ())),
        preferred_element_type=jnp.float32)
    out_ref[...] = (
        jax.lax.dot_general(rt_ref[...], tpre, (((1,), (0,)), ((), ())),
                            preferred_element_type=jnp.float32)
        + jax.lax.dot_general(rm_ref[...], mpre, (((1,), (0,)), ((), ())),
                              preferred_element_type=jnp.float32)
        + b1_ref[...])


def _prep(tissue_emb, metagene_emb, w1t, w1m, b1):
    nt, d = tissue_emb.shape
    nm = metagene_emb.shape[0]
    rows = jnp.arange(nt * nm, dtype=jnp.int32)
    rt = jax.nn.one_hot(rows // nm, nt, dtype=jnp.float32)
    rm = jax.nn.one_hot(rows % nm, nm, dtype=jnp.float32)
    return pl.pallas_call(
        _prep_body,
        out_shape=jax.ShapeDtypeStruct((nt * nm, d), jnp.float32),
    )(tissue_emb, metagene_emb, w1t, w1m, b1, rt, rm)


def _xp_body(x_ref, w_ref, out_ref):
    out_ref[...] = jax.lax.dot_general(
        x_ref[...], w_ref[...], (((1,), (0,)), ((), ())),
        preferred_element_type=jnp.float32)


def _xp(x, w1p):
    n, d = x.shape
    bn = 1000 if n % 1000 == 0 else n
    return pl.pallas_call(
        _xp_body,
        grid=(n // bn,),
        in_specs=[
            pl.BlockSpec((bn, d), lambda i: (i, 0)),
            pl.BlockSpec((d, d), lambda i: (0, 0)),
        ],
        out_specs=pl.BlockSpec((bn, d), lambda i: (i, 0)),
        out_shape=jax.ShapeDtypeStruct((n, d), jnp.float32),
        compiler_params=pltpu.CompilerParams(dimension_semantics=("parallel",)),
    )(x, w1p)


def _edge_body(xg_ref, gtm_ref, ea_ref, w1a_ref, w2_ref, b2_ref, we_ref,
               be_ref, msg_ref, eao_ref):
    ea = ea_ref[...]
    h = xg_ref[...] + gtm_ref[...] + jax.lax.dot_general(
        ea, w1a_ref[...], (((0,), (0,)), ((), ())),
        preferred_element_type=jnp.float32)
    h = jnp.maximum(h, 0.0)
    msg = jax.lax.dot_general(
        h, w2_ref[...], (((1,), (0,)), ((), ())),
        preferred_element_type=jnp.float32) + b2_ref[...]
    msg = jnp.maximum(msg, 0.0)
    msg_ref[...] = msg
    eao_ref[...] = ea + jax.lax.dot_general(
        we_ref[...], msg, (((0,), (1,)), ((), ())),
        preferred_element_type=jnp.float32) + be_ref[...]


def _edge(xg, gtm, ea_t, w1a, w2, b2r, we, ber, goff=0):
    e, d = xg.shape
    da = ea_t.shape[0]
    be_blk = BE if e % BE == 0 else e
    gob = goff // be_blk
    return pl.pallas_call(
        _edge_body,
        grid=(e // be_blk,),
        in_specs=[
            pl.BlockSpec((be_blk, d), lambda i: (i, 0)),
            pl.BlockSpec((be_blk, d), lambda i: (i + gob, 0)),
            pl.BlockSpec((da, be_blk), lambda i: (0, i)),
            pl.BlockSpec((da, d), lambda i: (0, 0)),
            pl.BlockSpec((d, d), lambda i: (0, 0)),
            pl.BlockSpec((1, d), lambda i: (0, 0)),
            pl.BlockSpec((d, da), lambda i: (0, 0)),
            pl.BlockSpec((da, 1), lambda i: (0, 0)),
        ],
        out_specs=[
            pl.BlockSpec((be_blk, d), lambda i: (i, 0)),
            pl.BlockSpec((da, be_blk), lambda i: (0, i)),
        ],
        out_shape=[
            jax.ShapeDtypeStruct((e, d), jnp.float32),
            jax.ShapeDtypeStruct((da, e), jnp.float32),
        ],
        compiler_params=pltpu.CompilerParams(dimension_semantics=("parallel",)),
    )(xg, gtm, ea_t, w1a, w2, b2r, we, ber)


def _upd_body(agg_ref, aggb_ref, x_ref, wu_ref, bu_ref, w1p_ref, xn_ref,
              xpn_ref):
    agg = (agg_ref[0] + agg_ref[1]) + (aggb_ref[0] + aggb_ref[1])
    upd = jax.lax.dot_general(
        agg, wu_ref[...], (((1,), (0,)), ((), ())),
        preferred_element_type=jnp.float32) + bu_ref[...]
    xn = x_ref[...] + jnp.maximum(upd, 0.0)
    xn_ref[...] = xn
    xpn_ref[...] = jax.lax.dot_general(
        xn, w1p_ref[...], (((1,), (0,)), ((), ())),
        preferred_element_type=jnp.float32)


def _upd(agg2, agg2b, x, wu, bur, w1p):
    n, d = x.shape
    bn = 1000 if n % 1000 == 0 else n
    return pl.pallas_call(
        _upd_body,
        grid=(n // bn,),
        in_specs=[
            pl.BlockSpec((2, bn, d), lambda i: (0, i, 0)),
            pl.BlockSpec((2, bn, d), lambda i: (0, i, 0)),
            pl.BlockSpec((bn, d), lambda i: (i, 0)),
            pl.BlockSpec((d, d), lambda i: (0, 0)),
            pl.BlockSpec((1, d), lambda i: (0, 0)),
            pl.BlockSpec((d, d), lambda i: (0, 0)),
        ],
        out_specs=[
            pl.BlockSpec((bn, d), lambda i: (i, 0)),
            pl.BlockSpec((bn, d), lambda i: (i, 0)),
        ],
        out_shape=[
            jax.ShapeDtypeStruct((n, d), jnp.float32),
            jax.ShapeDtypeStruct((n, d), jnp.float32),
        ],
        compiler_params=pltpu.CompilerParams(dimension_semantics=("parallel",)),
    )(agg2, agg2b, x, wu, bur, w1p)


def kernel(patient_idx, tissue_idx, metagene_idx, hyperedge_attr,
           patient_features, tissue_emb, metagene_emb,
           W1, b1, W2, b2, Wu, bu, We, be):
    e, da = hyperedge_attr.shape
    npat, d = patient_features.shape
    nm = metagene_emb.shape[0]
    w1p, w1t, w1m, w1a = W1[:d], W1[d:2 * d], W1[2 * d:3 * d], W1[3 * d:]
    pidx = patient_idx.astype(jnp.int32)
    cidx = (tissue_idx.astype(jnp.int32) * nm + metagene_idx.astype(jnp.int32))
    half = e // 2

    def _chunked(ix):
        return jnp.pad(ix.reshape(-1, CH), ((0, PAD_CH), (0, 0)))

    pidx2 = [_chunked(pidx[:half]), _chunked(pidx[half:])]
    cidx2 = _chunked(cidx)

    def _chunked_s(ix):
        return jnp.pad(ix.reshape(-1, CHS), ((0, PAD_CH), (0, 0)))

    pidx2s = [_chunked_s(pidx[:half]), _chunked_s(pidx[half:])]
    ea_t = [hyperedge_attr[:half].T, hyperedge_attr[half:].T]
    b1r = b1.reshape(1, d)
    b2r = b2.reshape(1, d)
    ber = be.reshape(da, 1)
    bur = bu.reshape(1, d)
    stripe = ((npat + 15) // 16 + 7) // 8 * 8
    zrows = jnp.zeros((stripe, d), jnp.float32)
    npad = stripe * 16

    ctab = _prep(tissue_emb, metagene_emb, w1t, w1m, b1r)
    gtm = _sc_gather(ctab, cidx2)
    x = patient_features
    xp = _xp(x, w1p)
    for _ in range(3):
        xg = [_sc_gather(xp, pidx2[0]), _sc_gather(xp, pidx2[1])]
        agg2 = [None, None]
        for hf in range(2):
            msg, ea_t[hf] = _edge(xg[hf], gtm, ea_t[hf], w1a, W2, b2r,
                                  We, ber, goff=hf * half)
            agg2[hf] = _sc_scatter(msg, pidx2s[hf], zrows)
        agg2 = [a.reshape(2, npad, d)[:, :npat] for a in agg2]
        x, xp = _upd(agg2[0], agg2[1], x, Wu, bur, w1p)
    return x
